# Initial kernel scaffold; baseline (speedup 1.0000x reference)
import jax
import jax.numpy as jnp
from jax.experimental import pallas as pl


def kernel(x, w, r):
    raise NotImplementedError("write your pallas kernel here")


# speedup vs baseline: 1.9248x; 1.9248x over previous
import functools

import jax
import jax.numpy as jnp
from jax import lax
from jax.experimental import pallas as pl
from jax.experimental.pallas import tpu as pltpu
from jax.experimental.pallas import tpu_sc as plsc

NCH = 2048
NTOK = 4096
TB = 512
NTB = NCH // TB
R = 128

_I32_BIG = 2**30


def _colsum_body(w_ref, out_ref):
    out_ref[...] = jnp.sum(w_ref[...], axis=0, keepdims=True)


def _sqcolsum_body(x_ref, out_ref):
    v = x_ref[...]
    out_ref[...] = jnp.sum(v * v, axis=0, keepdims=True)


def _scores_body(xa_ref, xb_ref, sa_ref, sb_ref, aa_ref, bb_ref,
                 scores_ref, nmin_ref, nidx_ref):
    j = pl.program_id(1)
    g = lax.dot_general(
        xa_ref[...], xb_ref[...], (((0,), (0,)), ((), ())),
        preferred_element_type=jnp.float32)
    s = sa_ref[...].reshape(TB, 1) + sb_ref[...]
    q = aa_ref[...].reshape(TB, 1) + bb_ref[...]
    t = s * jnp.sqrt(jnp.maximum(q - 2.0 * g, 0.0))
    sc = t * t
    scores_ref[...] = sc
    m = jnp.min(sc, axis=1, keepdims=True)
    col = lax.broadcasted_iota(jnp.int32, sc.shape, 1)
    idx = jnp.min(jnp.where(sc == m, col + j * TB, _I32_BIG), axis=1)
    mrow = m[:, 0]

    @pl.when(j == 0)
    def _():
        nmin_ref[0, :] = mrow
        nidx_ref[0, :] = idx

    @pl.when(j > 0)
    def _():
        cur = nmin_ref[0, :]
        cidx = nidx_ref[0, :]
        better = mrow < cur
        nmin_ref[0, :] = jnp.where(better, mrow, cur)
        nidx_ref[0, :] = jnp.where(better, idx, cidx)


def _select_body(nmin_hbm, nidx_hbm, src_hbm, dst_hbm, ms_hbm,
                 val_v, idx_v, summ_v, osrc_v, odst_v, oms_v, tmpv_v, tmpi_v):
    cid = lax.axis_index("c")
    sid = lax.axis_index("s")

    @pl.when((cid == 0) & (sid == 0))
    def _():
        pltpu.sync_copy(nmin_hbm, val_v)
        pltpu.sync_copy(nidx_hbm, idx_v)
        lane = lax.iota(jnp.int32, 16)
        mask0 = lane == 0
        inf = jnp.float32(jnp.inf)
        nblk = NCH // 16

        def pair_take(v2, i2, v1, i1):
            take = (v2 < v1) | ((v2 == v1) & (i2 < i1))
            return jnp.where(take, v2, v1), jnp.where(take, i2, i1)

        def bf_minpair(v, i):
            for sh in (8, 4, 2, 1):
                tmpv_v[...] = v
                tmpi_v[...] = i
                perm = lane ^ sh
                v2 = plsc.load_gather(tmpv_v, [perm])
                i2 = plsc.load_gather(tmpi_v, [perm])
                v, i = pair_take(v2, i2, v, i)
            return v, i

        def build(b, c):
            v = plsc.load_gather(val_v, [b * 16 + lane])
            mn, _ = bf_minpair(v, lane)
            plsc.store_scatter(summ_v, [jnp.full((16,), b, jnp.int32)],
                               mn, mask=mask0)
            return c

        lax.fori_loop(0, nblk, build, 0)

        def step(t, c):
            bestv = summ_v[pl.ds(0, 16)]
            besti = lane
            for q in range(1, nblk // 16):
                sv = summ_v[pl.ds(q * 16, 16)]
                bestv, besti = pair_take(sv, lane + q * 16, bestv, besti)
            mval, b = bf_minpair(bestv, besti)
            gidx = b * 16 + lane
            v = plsc.load_gather(val_v, [gidx])
            cand = jnp.where(v == mval, gidx, _I32_BIG)
            _, e = bf_minpair(v, cand)
            e = jnp.where(e == _I32_BIG, gidx, e)
            d = plsc.load_gather(idx_v, [e])
            tv = jnp.full((16,), t, jnp.int32)
            plsc.store_scatter(osrc_v, [tv], e, mask=mask0)
            plsc.store_scatter(odst_v, [tv], d, mask=mask0)
            plsc.store_scatter(oms_v, [tv], mval, mask=mask0)
            v2 = jnp.where(gidx == e, inf, v)
            plsc.store_scatter(val_v, [gidx], v2)
            mn2, _ = bf_minpair(v2, lane)
            plsc.store_scatter(summ_v, [b], mn2, mask=mask0)
            return c

        lax.fori_loop(0, R, step, 0)
        pltpu.sync_copy(osrc_v, src_hbm)
        pltpu.sync_copy(odst_v, dst_hbm)
        pltpu.sync_copy(oms_v, ms_hbm)


def _colsums(w):
    return pl.pallas_call(
        _colsum_body,
        grid=(8,),
        in_specs=[pl.BlockSpec((NTOK, 512), lambda k: (0, k))],
        out_specs=pl.BlockSpec((1, 512), lambda k: (0, k)),
        out_shape=jax.ShapeDtypeStruct((1, NTOK), jnp.float32),
    )(w)


def _sqcolsums(xh):
    return pl.pallas_call(
        _sqcolsum_body,
        grid=(4,),
        in_specs=[pl.BlockSpec((NTOK, 512), lambda k: (0, k))],
        out_specs=pl.BlockSpec((1, 512), lambda k: (0, k)),
        out_shape=jax.ShapeDtypeStruct((1, NCH), jnp.float32),
    )(xh)


def _scores_minidx(xa, xb, sa, sb, aa, bb):
    return pl.pallas_call(
        _scores_body,
        grid=(NTB, NTB),
        in_specs=[
            pl.BlockSpec((NTOK, TB), lambda i, j: (0, i)),
            pl.BlockSpec((NTOK, TB), lambda i, j: (0, j)),
            pl.BlockSpec((1, TB), lambda i, j: (0, i)),
            pl.BlockSpec((1, TB), lambda i, j: (0, j)),
            pl.BlockSpec((1, TB), lambda i, j: (0, i)),
            pl.BlockSpec((1, TB), lambda i, j: (0, j)),
        ],
        out_specs=[
            pl.BlockSpec((TB, TB), lambda i, j: (i, j)),
            pl.BlockSpec((1, TB), lambda i, j: (0, i)),
            pl.BlockSpec((1, TB), lambda i, j: (0, i)),
        ],
        out_shape=[
            jax.ShapeDtypeStruct((NCH, NCH), jnp.float32),
            jax.ShapeDtypeStruct((1, NCH), jnp.float32),
            jax.ShapeDtypeStruct((1, NCH), jnp.int32),
        ],
    )(xa, xb, sa, sb, aa, bb)


@functools.lru_cache(maxsize=1)
def _select_topr():
    return functools.partial(
        pl.kernel,
        mesh=plsc.VectorSubcoreMesh(core_axis_name="c", subcore_axis_name="s"),
        compiler_params=pltpu.CompilerParams(needs_layout_passes=False),
        out_type=[
            jax.ShapeDtypeStruct((R,), jnp.int32),
            jax.ShapeDtypeStruct((R,), jnp.int32),
            jax.ShapeDtypeStruct((R,), jnp.float32),
        ],
        scratch_types=[
            pltpu.VMEM((NCH,), jnp.float32),
            pltpu.VMEM((NCH,), jnp.int32),
            pltpu.VMEM((NCH // 16,), jnp.float32),
            pltpu.VMEM((R,), jnp.int32),
            pltpu.VMEM((R,), jnp.int32),
            pltpu.VMEM((R,), jnp.float32),
            pltpu.VMEM((16,), jnp.float32),
            pltpu.VMEM((16,), jnp.int32),
        ],
    )(_select_body)


def kernel(x, w, r):
    del r
    xf = x.reshape(NTOK, NTOK)
    xa = xf[:, 0::2]
    xb = xf[:, 1::2]
    ws = _colsums(w)
    sa = ws[:, 0::2]
    sb = ws[:, 1::2]
    aa = _sqcolsums(xa)
    bb = _sqcolsums(xb)
    scores, nmin, nidx = _scores_minidx(xa, xb, sa, sb, aa, bb)
    src_idx, dst_idx, matched = _select_topr()(nmin.reshape(NCH),
                                               nidx.reshape(NCH))
    return scores, src_idx, dst_idx, matched


# speedup vs baseline: 22.7521x; 11.8204x over previous
import functools

import jax
import jax.numpy as jnp
from jax import lax
from jax.experimental import pallas as pl
from jax.experimental.pallas import tpu as pltpu
from jax.experimental.pallas import tpu_sc as plsc

NCH = 2048
NTOK = 4096
TB = 512
NTB = NCH // TB
R = 128

_I32_BIG = 2**30


def _deint_body(xf_ref, xa_ref, xb_ref, aa_ref, bb_ref):
    v = xf_ref[...]
    row = lax.broadcasted_iota(jnp.int32, (512, 256), 0)
    col = lax.broadcasted_iota(jnp.int32, (512, 256), 1)
    pe = (row == 2 * col).astype(jnp.bfloat16)
    po = (row == 2 * col + 1).astype(jnp.bfloat16)
    vh = v.astype(jnp.bfloat16)
    r1 = v - vh.astype(jnp.float32)
    vm = r1.astype(jnp.bfloat16)
    vl = (r1 - vm.astype(jnp.float32)).astype(jnp.bfloat16)

    def sel(p):
        dn = (((1,), (0,)), ((), ()))
        return (lax.dot_general(vh, p, dn, preferred_element_type=jnp.float32)
                + lax.dot_general(vm, p, dn,
                                  preferred_element_type=jnp.float32)
                + lax.dot_general(vl, p, dn,
                                  preferred_element_type=jnp.float32))

    xa = sel(pe)
    xb = sel(po)
    xa_ref[...] = xa
    xb_ref[...] = xb
    aa_ref[...] = jnp.sum(xa * xa, axis=0, keepdims=True)
    bb_ref[...] = jnp.sum(xb * xb, axis=0, keepdims=True)


def _colsum_body(w_ref, out_ref):
    out_ref[...] = jnp.sum(w_ref[...], axis=0, keepdims=True)


def _sqcolsum_body(x_ref, out_ref):
    v = x_ref[...]
    out_ref[...] = jnp.sum(v * v, axis=0, keepdims=True)


def _scores_body(xa_ref, xb_ref, sa_ref, sb_ref, aa_ref, bb_ref,
                 scores_ref, nmin_ref, nidx_ref):
    j = pl.program_id(1)
    g = lax.dot_general(
        xa_ref[...], xb_ref[...], (((0,), (0,)), ((), ())),
        preferred_element_type=jnp.float32)
    s = sa_ref[...].reshape(TB, 1) + sb_ref[...]
    q = aa_ref[...].reshape(TB, 1) + bb_ref[...]
    t = s * jnp.sqrt(jnp.maximum(q - 2.0 * g, 0.0))
    sc = t * t
    scores_ref[...] = sc
    m = jnp.min(sc, axis=1, keepdims=True)
    col = lax.broadcasted_iota(jnp.int32, sc.shape, 1)
    idx = jnp.min(jnp.where(sc == m, col + j * TB, _I32_BIG), axis=1)
    mrow = m[:, 0]

    @pl.when(j == 0)
    def _():
        nmin_ref[0, :] = mrow
        nidx_ref[0, :] = idx

    @pl.when(j > 0)
    def _():
        cur = nmin_ref[0, :]
        cidx = nidx_ref[0, :]
        better = mrow < cur
        nmin_ref[0, :] = jnp.where(better, mrow, cur)
        nidx_ref[0, :] = jnp.where(better, idx, cidx)


def _select_body(nmin_hbm, nidx_hbm, src_hbm, dst_hbm, ms_hbm,
                 val_v, idx_v, summ_v, osrc_v, odst_v, oms_v, tmpv_v, tmpi_v):
    cid = lax.axis_index("c")
    sid = lax.axis_index("s")

    @pl.when((cid == 0) & (sid == 0))
    def _():
        pltpu.sync_copy(nmin_hbm, val_v)
        pltpu.sync_copy(nidx_hbm, idx_v)
        lane = lax.iota(jnp.int32, 16)
        mask0 = lane == 0
        inf = jnp.float32(jnp.inf)
        nblk = NCH // 16

        def pair_take(v2, i2, v1, i1):
            take = (v2 < v1) | ((v2 == v1) & (i2 < i1))
            return jnp.where(take, v2, v1), jnp.where(take, i2, i1)

        def bf_minpair(v, i):
            for sh in (8, 4, 2, 1):
                tmpv_v[...] = v
                tmpi_v[...] = i
                perm = lane ^ sh
                v2 = plsc.load_gather(tmpv_v, [perm])
                i2 = plsc.load_gather(tmpi_v, [perm])
                v, i = pair_take(v2, i2, v, i)
            return v, i

        def build(b, c):
            v = plsc.load_gather(val_v, [b * 16 + lane])
            mn, _ = bf_minpair(v, lane)
            plsc.store_scatter(summ_v, [jnp.full((16,), b, jnp.int32)],
                               mn, mask=mask0)
            return c

        lax.fori_loop(0, nblk, build, 0)

        def step(t, c):
            bestv = summ_v[pl.ds(0, 16)]
            besti = lane
            for q in range(1, nblk // 16):
                sv = summ_v[pl.ds(q * 16, 16)]
                bestv, besti = pair_take(sv, lane + q * 16, bestv, besti)
            mval, b = bf_minpair(bestv, besti)
            gidx = b * 16 + lane
            v = plsc.load_gather(val_v, [gidx])
            cand = jnp.where(v == mval, gidx, _I32_BIG)
            _, e = bf_minpair(v, cand)
            e = jnp.where(e == _I32_BIG, gidx, e)
            d = plsc.load_gather(idx_v, [e])
            tv = jnp.full((16,), t, jnp.int32)
            plsc.store_scatter(osrc_v, [tv], e, mask=mask0)
            plsc.store_scatter(odst_v, [tv], d, mask=mask0)
            plsc.store_scatter(oms_v, [tv], mval, mask=mask0)
            v2 = jnp.where(gidx == e, inf, v)
            plsc.store_scatter(val_v, [gidx], v2)
            mn2, _ = bf_minpair(v2, lane)
            plsc.store_scatter(summ_v, [b], mn2, mask=mask0)
            return c

        lax.fori_loop(0, R, step, 0)
        pltpu.sync_copy(osrc_v, src_hbm)
        pltpu.sync_copy(odst_v, dst_hbm)
        pltpu.sync_copy(oms_v, ms_hbm)


def _deint(xf):
    return pl.pallas_call(
        _deint_body,
        grid=(8,),
        in_specs=[pl.BlockSpec((NTOK, 512), lambda k: (0, k))],
        out_specs=[
            pl.BlockSpec((NTOK, 256), lambda k: (0, k)),
            pl.BlockSpec((NTOK, 256), lambda k: (0, k)),
            pl.BlockSpec((1, 256), lambda k: (0, k)),
            pl.BlockSpec((1, 256), lambda k: (0, k)),
        ],
        out_shape=[
            jax.ShapeDtypeStruct((NTOK, NCH), jnp.float32),
            jax.ShapeDtypeStruct((NTOK, NCH), jnp.float32),
            jax.ShapeDtypeStruct((1, NCH), jnp.float32),
            jax.ShapeDtypeStruct((1, NCH), jnp.float32),
        ],
    )(xf)


def _colsums(w):
    return pl.pallas_call(
        _colsum_body,
        grid=(4,),
        in_specs=[pl.BlockSpec((NTOK, 1024), lambda k: (0, k))],
        out_specs=pl.BlockSpec((1, 1024), lambda k: (0, k)),
        out_shape=jax.ShapeDtypeStruct((1, NTOK), jnp.float32),
    )(w)


def _sqcolsums(xh):
    return pl.pallas_call(
        _sqcolsum_body,
        grid=(4,),
        in_specs=[pl.BlockSpec((NTOK, 512), lambda k: (0, k))],
        out_specs=pl.BlockSpec((1, 512), lambda k: (0, k)),
        out_shape=jax.ShapeDtypeStruct((1, NCH), jnp.float32),
    )(xh)


def _scores_minidx(xa, xb, sa, sb, aa, bb):
    return pl.pallas_call(
        _scores_body,
        grid=(NTB, NTB),
        in_specs=[
            pl.BlockSpec((NTOK, TB), lambda i, j: (0, i)),
            pl.BlockSpec((NTOK, TB), lambda i, j: (0, j)),
            pl.BlockSpec((1, TB), lambda i, j: (0, i)),
            pl.BlockSpec((1, TB), lambda i, j: (0, j)),
            pl.BlockSpec((1, TB), lambda i, j: (0, i)),
            pl.BlockSpec((1, TB), lambda i, j: (0, j)),
        ],
        out_specs=[
            pl.BlockSpec((TB, TB), lambda i, j: (i, j)),
            pl.BlockSpec((1, TB), lambda i, j: (0, i)),
            pl.BlockSpec((1, TB), lambda i, j: (0, i)),
        ],
        out_shape=[
            jax.ShapeDtypeStruct((NCH, NCH), jnp.float32),
            jax.ShapeDtypeStruct((1, NCH), jnp.float32),
            jax.ShapeDtypeStruct((1, NCH), jnp.int32),
        ],
    )(xa, xb, sa, sb, aa, bb)


@functools.lru_cache(maxsize=1)
def _select_topr():
    return functools.partial(
        pl.kernel,
        mesh=plsc.VectorSubcoreMesh(core_axis_name="c", subcore_axis_name="s"),
        compiler_params=pltpu.CompilerParams(needs_layout_passes=False),
        out_type=[
            jax.ShapeDtypeStruct((R,), jnp.int32),
            jax.ShapeDtypeStruct((R,), jnp.int32),
            jax.ShapeDtypeStruct((R,), jnp.float32),
        ],
        scratch_types=[
            pltpu.VMEM((NCH,), jnp.float32),
            pltpu.VMEM((NCH,), jnp.int32),
            pltpu.VMEM((NCH // 16,), jnp.float32),
            pltpu.VMEM((R,), jnp.int32),
            pltpu.VMEM((R,), jnp.int32),
            pltpu.VMEM((R,), jnp.float32),
            pltpu.VMEM((16,), jnp.float32),
            pltpu.VMEM((16,), jnp.int32),
        ],
    )(_select_body)


def kernel(x, w, r):
    del r
    xf = x.reshape(NTOK, NTOK)
    xa, xb, aa, bb = _deint(xf)
    ws = _colsums(w)
    sa = ws[:, 0::2]
    sb = ws[:, 1::2]
    scores, nmin, nidx = _scores_minidx(xa, xb, sa, sb, aa, bb)
    src_idx, dst_idx, matched = _select_topr()(nmin.reshape(NCH),
                                               nidx.reshape(NCH))
    return scores, src_idx, dst_idx, matched
